# Initial kernel scaffold; baseline (speedup 1.0000x reference)
import jax
import jax.numpy as jnp
from jax.experimental import pallas as pl


def kernel(history_embedding_multivariate, seq_positions_multivariate, seq_length):
    raise NotImplementedError("write your pallas kernel here")


# speedup vs baseline: 16.0836x; 16.0836x over previous
import functools

import jax
import jax.numpy as jnp
from jax import lax
from jax.experimental import pallas as pl
from jax.experimental.pallas import tpu as pltpu
from jax.experimental.pallas import tpu_sc as plsc

_B, _E, _S, _D = 16, 20, 2048, 32
_ES = _E * _S
_HALF = _ES // 2
_CHUNKS = _HALF // 16
_SLOTS_W = _S // 2
_CHUNK = 256
_CHUNK_ROWS = _CHUNK


def _shift_up(v):
    ids = jnp.minimum(lax.iota(jnp.int32, 16) + 1, 15)
    return v.at[ids].get(mode="promise_in_bounds")


def _body(h_hbm, pos_hbm, out_hbm, pos_v, table_v, part_v, sgidx_v, p_v,
          super_v, outbuf_v, shared, sem):
    c = lax.axis_index("c")
    s = lax.axis_index("s")
    b = c * 8 + s // 2
    half = s % 2
    iot = lax.iota(jnp.int32, 16)

    pos_start = b * _ES + half * _HALF
    pltpu.sync_copy(pos_hbm.at[pl.ds(pos_start, _HALF)], pos_v)

    neg1 = jnp.full((16,), -1, jnp.int32)
    for k in range(_S // 16):
        table_v[pl.ds(k * 16, 16)] = neg1

    i0 = half * _HALF

    def chunk_body(j, carry):
        base = j * 16
        idx = pos_v[pl.ds(base, 16)]
        packed = idx * 65536 + (i0 + base + iot)
        srt, _ = plsc.sort_key_val(packed, packed)
        spos = srt >> 16
        sval = srt & 0xFFFF
        last = (spos != _shift_up(spos)) | (iot == 15)
        plsc.store_scatter(table_v, [spos], sval, mask=last)
        return carry

    lax.fori_loop(0, _CHUNKS, chunk_body, jnp.int32(0))

    pltpu.sync_copy(table_v, shared.at[s])
    plsc.subcore_barrier()
    pltpu.sync_copy(shared.at[s ^ 1], part_v)

    slot0 = half * _SLOTS_W

    def merge_body(j, carry):
        t = jnp.maximum(table_v[pl.ds(slot0 + j * 16, 16)],
                        part_v[pl.ds(slot0 + j * 16, 16)])
        g = b * _ES + jnp.maximum(t, 0)
        sgidx_v[pl.ds(j * 16, 16)] = g >> 2
        jl = (j * 16 + iot) & (_CHUNK - 1)
        p_v[pl.ds(j * 16, 16)] = jnp.where(t < 0, _CHUNK_ROWS * 4,
                                           jl * 4 + (g & 3))
        return carry

    lax.fori_loop(0, _SLOTS_W // 16, merge_body, jnp.int32(0))

    zero = jnp.zeros((16,), jnp.float32)
    for k in range(128 // 16):
        super_v[_CHUNK_ROWS, pl.ds(k * 16, 16)] = zero

    for c in range(_SLOTS_W // _CHUNK):
        pltpu.async_copy(h_hbm.at[sgidx_v.at[pl.ds(c * _CHUNK, _CHUNK)]],
                         super_v.at[pl.ds(0, _CHUNK)], sem).wait()

        def extract_body(g16, carry, c=c):
            pv = p_v[pl.ds(c * _CHUNK + g16 * 16, 16)]
            lr = pv >> 2
            off = (pv & 3) * 32
            oq = (g16 * 16 + iot) >> 2
            orow = ((g16 * 16 + iot) & 3) * 32
            for c2 in range(_D):
                vals = plsc.load_gather(super_v, [lr, off + c2])
                plsc.store_scatter(outbuf_v, [oq, orow + c2], vals)
            return carry

        lax.fori_loop(0, _CHUNK // 16, extract_body, jnp.int32(0))
        out_base = pl.multiple_of((b * _S + half * _SLOTS_W + c * _CHUNK) // 4,
                                  _CHUNK // 4)
        pltpu.sync_copy(outbuf_v,
                        out_hbm.at[pl.ds(out_base, _CHUNK // 4)])


@jax.jit
def _realign(h_flat, pos_flat):
    mesh = plsc.VectorSubcoreMesh(core_axis_name="c", subcore_axis_name="s")
    return pl.kernel(
        _body,
        mesh=mesh,
        compiler_params=pltpu.CompilerParams(needs_layout_passes=False),
        out_type=jax.ShapeDtypeStruct((_B * _S * _D // 128, 128),
                                      jnp.float32),
        scratch_types=[
            pltpu.VMEM((_HALF,), jnp.int32),
            pltpu.VMEM((_S,), jnp.int32),
            pltpu.VMEM((_S,), jnp.int32),
            pltpu.VMEM((_SLOTS_W,), jnp.int32),
            pltpu.VMEM((_SLOTS_W,), jnp.int32),
            pltpu.VMEM((_CHUNK_ROWS + 8, 128), jnp.float32),
            pltpu.VMEM((_CHUNK * _D // 128, 128), jnp.float32),
            pltpu.VMEM_SHARED((16, _S), jnp.int32),
            pltpu.SemaphoreType.DMA,
        ],
    )(h_flat, pos_flat)


def kernel(history_embedding_multivariate, seq_positions_multivariate,
           seq_length):
    del seq_length
    h = history_embedding_multivariate
    B, E, S, D = h.shape
    h_flat = h.reshape(B * E * S * D // 128, 128)
    pos_flat = seq_positions_multivariate.astype(jnp.int32).reshape(B * E * S)
    out = _realign(h_flat, pos_flat)
    return out.reshape(B, S, D)


# speedup vs baseline: 16.4122x; 1.0204x over previous
import functools

import jax
import jax.numpy as jnp
from jax import lax
from jax.experimental import pallas as pl
from jax.experimental.pallas import tpu as pltpu
from jax.experimental.pallas import tpu_sc as plsc

_B, _E, _S, _D = 16, 20, 2048, 32
_ES = _E * _S
_HALF = _ES // 2
_CHUNKS = _HALF // 16
_SLOTS_W = _S // 2
_CHUNK = 256
_CHUNK_ROWS = _CHUNK


def _shift_up(v):
    ids = jnp.minimum(lax.iota(jnp.int32, 16) + 1, 15)
    return v.at[ids].get(mode="promise_in_bounds")


def _body(h_hbm, pos_hbm, out_hbm, pos_v, table_v, part_v, sgidx_v, p_v,
          super_v, outbuf_v, shared, sem):
    c = lax.axis_index("c")
    s = lax.axis_index("s")
    b = c * 8 + s // 2
    half = s % 2
    iot = lax.iota(jnp.int32, 16)

    pos_start = b * _ES + half * _HALF
    pltpu.sync_copy(pos_hbm.at[pl.ds(pos_start, _HALF)], pos_v)

    neg1 = jnp.full((16,), -1, jnp.int32)
    for k in range(_S // 16):
        table_v[pl.ds(k * 16, 16)] = neg1

    i0 = half * _HALF

    def chunk_body(j, carry):
        base = j * 16
        idx = pos_v[pl.ds(base, 16)]
        packed = idx * 65536 + (i0 + base + iot)
        srt, _ = plsc.sort_key_val(packed, packed)
        spos = srt >> 16
        sval = srt & 0xFFFF
        last = (spos != _shift_up(spos)) | (iot == 15)
        plsc.store_scatter(table_v, [spos], sval, mask=last)
        return carry

    lax.fori_loop(0, _CHUNKS, chunk_body, jnp.int32(0))

    pltpu.sync_copy(table_v, shared.at[s])
    plsc.subcore_barrier()
    pltpu.sync_copy(shared.at[s ^ 1], part_v)

    slot0 = half * _SLOTS_W

    def merge_body(j, carry):
        t = jnp.maximum(table_v[pl.ds(slot0 + j * 16, 16)],
                        part_v[pl.ds(slot0 + j * 16, 16)])
        g = b * _ES + jnp.maximum(t, 0)
        sgidx_v[pl.ds(j * 16, 16)] = g >> 2
        jl = (j * 16 + iot) & (_CHUNK - 1)
        p_v[pl.ds(j * 16, 16)] = jnp.where(t < 0, _CHUNK_ROWS * 4,
                                           jl * 4 + (g & 3))
        return carry

    lax.fori_loop(0, _SLOTS_W // 16, merge_body, jnp.int32(0))

    zero = jnp.zeros((16,), jnp.float32)
    for k in range(128 // 16):
        super_v[_CHUNK_ROWS, pl.ds(k * 16, 16)] = zero

    for c in range(_SLOTS_W // _CHUNK):
        pltpu.async_copy(h_hbm.at[sgidx_v.at[pl.ds(c * _CHUNK, _CHUNK)]],
                         super_v.at[pl.ds(0, _CHUNK)], sem).wait()

        def extract_body(g16, carry, c=c):
            pv = p_v[pl.ds(c * _CHUNK + g16 * 16, 16)]
            lr = pv >> 2
            off = (pv & 3) * 32
            orow = g16 * 16 + iot
            for c2 in range(_D):
                vals = plsc.load_gather(super_v, [lr, off + c2])
                plsc.store_scatter(outbuf_v, [orow, jnp.full((16,), c2,
                                                             jnp.int32)],
                                   vals)
            return carry

        lax.fori_loop(0, _CHUNK // 16, extract_body, jnp.int32(0))
        out_base = pl.multiple_of(b * _S + half * _SLOTS_W + c * _CHUNK,
                                  _CHUNK)
        pltpu.sync_copy(outbuf_v, out_hbm.at[pl.ds(out_base, _CHUNK)])


@jax.jit
def _realign(h_flat, pos_flat):
    mesh = plsc.VectorSubcoreMesh(core_axis_name="c", subcore_axis_name="s")
    return pl.kernel(
        _body,
        mesh=mesh,
        compiler_params=pltpu.CompilerParams(needs_layout_passes=False),
        out_type=jax.ShapeDtypeStruct((_B * _S, _D), jnp.float32),
        scratch_types=[
            pltpu.VMEM((_HALF,), jnp.int32),
            pltpu.VMEM((_S,), jnp.int32),
            pltpu.VMEM((_S,), jnp.int32),
            pltpu.VMEM((_SLOTS_W,), jnp.int32),
            pltpu.VMEM((_SLOTS_W,), jnp.int32),
            pltpu.VMEM((_CHUNK_ROWS + 8, 128), jnp.float32),
            pltpu.VMEM((_CHUNK, _D), jnp.float32),
            pltpu.VMEM_SHARED((16, _S), jnp.int32),
            pltpu.SemaphoreType.DMA,
        ],
    )(h_flat, pos_flat)


def kernel(history_embedding_multivariate, seq_positions_multivariate,
           seq_length):
    del seq_length
    h = history_embedding_multivariate
    B, E, S, D = h.shape
    h_flat = h.reshape(B * E * S * D // 128, 128)
    pos_flat = seq_positions_multivariate.astype(jnp.int32).reshape(B * E * S)
    out = _realign(h_flat, pos_flat)
    return out.reshape(B, S, D)


# speedup vs baseline: 39.9478x; 2.4340x over previous
import functools

import jax
import jax.numpy as jnp
from jax import lax
from jax.experimental import pallas as pl
from jax.experimental.pallas import tpu as pltpu
from jax.experimental.pallas import tpu_sc as plsc

_B, _E, _S, _D = 16, 20, 2048, 32
_ES = _E * _S
_CHUNKS2 = _E * (_S // 2) // 32
_BPC = _B // 2
_BSTRIDE = _E * _D * _S
_NIDX = 16 * _S


def _shift_up(v):
    ids = jnp.minimum(lax.iota(jnp.int32, 16) + 1, 15)
    return v.at[ids].get(mode="promise_in_bounds")


def _body(h_hbm, pos_hbm, out_hbm, pos_v, table_v, lo_v, hi_v, base_v,
          strd_v, validf_v, idx_v, vals_v, shared_t, sem):
    c = lax.axis_index("c")
    s = lax.axis_index("s")
    b = c * _BPC + s // 2
    half = s % 2
    iot = lax.iota(jnp.int32, 16)

    pltpu.sync_copy(pos_hbm.at[:, b, pl.ds(half * (_S // 2), _S // 2)],
                    pos_v)

    neg1 = jnp.full((16,), -1, jnp.int32)
    for k in range(_S // 16):
        table_v[pl.ds(k * 16, 16)] = neg1

    i0 = half * (_S // 2)

    def chunk_body(j, carry):
        for u in range(2):
            jj = j * 2 + u
            e = jj >> 6
            sc = jj & 63
            idx = pos_v[e, pl.ds(sc * 16, 16)]
            packed = idx * 65536 + (e * _S + i0 + sc * 16 + iot)
            srt, _ = plsc.sort_key_val(packed, packed)
            spos = srt >> 16
            sval = srt & 0xFFFF
            last = (spos != _shift_up(spos)) | (iot == 15)
            plsc.store_scatter(table_v, [spos], sval, mask=last)
        return carry

    lax.fori_loop(0, _CHUNKS2, chunk_body, jnp.int32(0))

    pltpu.sync_copy(table_v, shared_t.at[s])
    plsc.subcore_barrier()
    pltpu.sync_copy(shared_t.at[(s // 2) * 2], lo_v)
    pltpu.sync_copy(shared_t.at[(s // 2) * 2 + 1], hi_v)

    bbase = b * _BSTRIDE

    def merge_body(j, n_inv):
        t = jnp.maximum(lo_v[pl.ds(j * 16, 16)], hi_v[pl.ds(j * 16, 16)])
        tc = jnp.maximum(t, 0)
        base_v[pl.ds(j * 16, 16)] = (bbase + (tc >> 11) * (_D * _S)
                                     + (tc & (_S - 1)))
        inv = t < 0
        strd_v[pl.ds(j * 16, 16)] = jnp.where(inv, 0, _S)
        validf_v[pl.ds(j * 16, 16)] = jnp.where(inv, jnp.float32(0.0),
                                                jnp.float32(1.0))
        return n_inv + jnp.sum(inv.astype(jnp.int32))

    n_inv = lax.fori_loop(0, _S // 16, merge_body, jnp.int32(0))

    d0 = half * 16

    def bld_body(g, carry):
        base = base_v[pl.ds(g * 16, 16)]
        strd = strd_v[pl.ds(g * 16, 16)]
        v = base + d0 * strd
        for di in range(16):
            idx_v[pl.ds(di * _S + g * 16, 16)] = v
            v = v + strd
        return carry

    lax.fori_loop(0, _S // 16, bld_body, jnp.int32(0))

    pltpu.async_copy(h_hbm.at[idx_v], vals_v, sem).wait()

    @pl.when(n_inv > 0)
    def _fix():
        def fix_body(v, carry):
            m = validf_v[pl.ds((v & 127) * 16, 16)]
            vals_v[pl.ds(v * 16, 16)] = vals_v[pl.ds(v * 16, 16)] * m
            return carry

        lax.fori_loop(0, _NIDX // 16, fix_body, jnp.int32(0))

    for di in range(16):
        pltpu.sync_copy(
            vals_v.at[pl.ds(di * _S, _S)],
            out_hbm.at[pl.ds((b * _D + d0 + di) * _S, _S)])


@jax.jit
def _realign(h_flat, pos_t):
    mesh = plsc.VectorSubcoreMesh(core_axis_name="c", subcore_axis_name="s")
    return pl.kernel(
        _body,
        mesh=mesh,
        compiler_params=pltpu.CompilerParams(needs_layout_passes=False),
        out_type=jax.ShapeDtypeStruct((_B * _D * _S,), jnp.float32),
        scratch_types=[
            pltpu.VMEM((_E, _S // 2), jnp.int32),
            pltpu.VMEM((_S,), jnp.int32),
            pltpu.VMEM((_S,), jnp.int32),
            pltpu.VMEM((_S,), jnp.int32),
            pltpu.VMEM((_S,), jnp.int32),
            pltpu.VMEM((_S,), jnp.int32),
            pltpu.VMEM((_S,), jnp.float32),
            pltpu.VMEM((_NIDX,), jnp.int32),
            pltpu.VMEM((_NIDX,), jnp.float32),
            pltpu.VMEM_SHARED((16, _S), jnp.int32),
            pltpu.SemaphoreType.DMA,
        ],
    )(h_flat, pos_t)


def kernel(history_embedding_multivariate, seq_positions_multivariate,
           seq_length):
    del seq_length
    h = history_embedding_multivariate
    B, E, S, D = h.shape
    h_flat = h.transpose(0, 1, 3, 2).reshape(B * E * D * S)
    pos_t = seq_positions_multivariate.astype(jnp.int32).transpose(1, 0, 2)
    out = _realign(h_flat, pos_t)
    return out.reshape(B, D, S).transpose(0, 2, 1)


# speedup vs baseline: 71.4925x; 1.7896x over previous
import functools

import jax
import jax.numpy as jnp
from jax import lax
from jax.experimental import pallas as pl
from jax.experimental.pallas import tpu as pltpu
from jax.experimental.pallas import tpu_sc as plsc

_B, _E, _S, _D = 16, 20, 2048, 32
_ES = _E * _S
_CHUNKS2 = _E * (_S // 2) // 32
_BPC = _B // 2
_BSTRIDE = _E * _D * _S
_NIDX = 16 * _S


def _shift_up(v):
    ids = jnp.minimum(lax.iota(jnp.int32, 16) + 1, 15)
    return v.at[ids].get(mode="promise_in_bounds")


def _body(h_hbm, pos_hbm, out_hbm, pos_v, table_v, lo_v, hi_v, base_v,
          validf_v, idx_v, vals_v, shared_t, sem):
    c = lax.axis_index("c")
    s = lax.axis_index("s")
    b = c * _BPC + s // 2
    half = s % 2
    iot = lax.iota(jnp.int32, 16)

    pltpu.sync_copy(pos_hbm.at[:, b, pl.ds(half * (_S // 2), _S // 2)],
                    pos_v)

    neg1 = jnp.full((16,), -1, jnp.int32)
    for k in range(_S // 16):
        table_v[pl.ds(k * 16, 16)] = neg1

    i0 = half * (_S // 2)

    def chunk_body(j, carry):
        for u in range(2):
            jj = j * 2 + u
            e = jj >> 6
            sc = jj & 63
            idx = pos_v[e, pl.ds(sc * 16, 16)]
            packed = idx * 65536 + (e * _S + i0 + sc * 16 + iot)
            srt, _ = plsc.sort_key_val(packed, packed)
            spos = srt >> 16
            sval = srt & 0xFFFF
            last = (spos != _shift_up(spos)) | (iot == 15)
            plsc.store_scatter(table_v, [spos], sval, mask=last)
        return carry

    lax.fori_loop(0, _CHUNKS2, chunk_body, jnp.int32(0))

    pltpu.sync_copy(table_v, shared_t.at[s])
    plsc.subcore_barrier()
    pltpu.sync_copy(shared_t.at[(s // 2) * 2], lo_v)
    pltpu.sync_copy(shared_t.at[(s // 2) * 2 + 1], hi_v)

    bbase = b * _BSTRIDE

    def merge_body(j, n_inv):
        t = jnp.maximum(lo_v[pl.ds(j * 16, 16)], hi_v[pl.ds(j * 16, 16)])
        tc = jnp.maximum(t, 0)
        sw = tc & (_S - 1)
        base_v[pl.ds(j * 16, 16)] = (bbase + (tc >> 11) * (_D * _S)
                                     + (sw >> 7) * 1024 + (sw & 127))
        inv = t < 0
        validf_v[pl.ds(j * 16, 16)] = jnp.where(inv, jnp.float32(0.0),
                                                jnp.float32(1.0))
        return n_inv + jnp.sum(inv.astype(jnp.int32))

    n_inv = lax.fori_loop(0, _S // 16, merge_body, jnp.int32(0))

    def bld_body(g, carry):
        base = base_v[pl.ds(g * 16, 16)]
        tb = (g >> 3) * 1024 + (g & 7) * 16
        for dhl in range(2):
            for dlo in range(8):
                dphys = (half * 2 + dhl) * (_D * _S // 4) + dlo * 128
                idx_v[pl.ds(tb + dhl * 16384 + dlo * 128, 16)] = base + dphys
        return carry

    lax.fori_loop(0, _S // 16, bld_body, jnp.int32(0))

    pltpu.async_copy(h_hbm.at[idx_v], vals_v, sem).wait()

    @pl.when(n_inv > 0)
    def _fix():
        def fix_body(v, carry):
            m = validf_v[pl.ds(((v >> 6) & 15) * 128 + (v & 7) * 16, 16)]
            vals_v[pl.ds(v * 16, 16)] = vals_v[pl.ds(v * 16, 16)] * m
            return carry

        lax.fori_loop(0, _NIDX // 16, fix_body, jnp.int32(0))

    for dhl in range(2):
        pltpu.sync_copy(
            vals_v.at[pl.ds(dhl * 16384, 16384)],
            out_hbm.at[pl.ds(b * (_D * _S) + (half * 2 + dhl) * 16384,
                             16384)])


@jax.jit
def _realign(h_flat, pos_t):
    mesh = plsc.VectorSubcoreMesh(core_axis_name="c", subcore_axis_name="s")
    return pl.kernel(
        _body,
        mesh=mesh,
        compiler_params=pltpu.CompilerParams(needs_layout_passes=False),
        out_type=jax.ShapeDtypeStruct((_B * _D * _S,), jnp.float32),
        scratch_types=[
            pltpu.VMEM((_E, _S // 2), jnp.int32),
            pltpu.VMEM((_S,), jnp.int32),
            pltpu.VMEM((_S,), jnp.int32),
            pltpu.VMEM((_S,), jnp.int32),
            pltpu.VMEM((_S,), jnp.int32),
            pltpu.VMEM((_S,), jnp.float32),
            pltpu.VMEM((_NIDX,), jnp.int32),
            pltpu.VMEM((_NIDX,), jnp.float32),
            pltpu.VMEM_SHARED((16, _S), jnp.int32),
            pltpu.SemaphoreType.DMA,
        ],
    )(h_flat, pos_t)


def kernel(history_embedding_multivariate, seq_positions_multivariate,
           seq_length):
    del seq_length
    h = history_embedding_multivariate
    B, E, S, D = h.shape
    h_flat = (h.reshape(B, E, S // 128, 128, D // 8, 8)
              .transpose(0, 1, 4, 2, 5, 3).reshape(B * E * D * S))
    pos_t = seq_positions_multivariate.astype(jnp.int32).transpose(1, 0, 2)
    out = _realign(h_flat, pos_t)
    return (out.reshape(B, D // 8, S // 128, 8, 128)
            .transpose(0, 2, 4, 1, 3).reshape(B, S, D))
